# baseline (speedup 1.0000x reference)
import functools

import jax
import jax.numpy as jnp
from jax import lax
from jax.experimental import pallas as pl
from jax.experimental.pallas import tpu as pltpu
from jax.experimental.pallas import tpu_sc as plsc

N = 10000
E = 160000
D = 300
L = 5
G = 64
FD = 512

NC = 2
NS = 16
DH = 80
NQ = 4
CH = 128
CPT = 80
CPTC = 40
EP = NS * CPT * CH
ROWS = 10240
RPT = ROWS // NS
DUMMY = N + 16


@functools.cache
def _mesh():
    return plsc.VectorSubcoreMesh(core_axis_name="c", subcore_axis_name="s",
                                  num_cores=NC, num_subcores=NS)


def _writeout(acc, buf, out, src_base, dst_base):
    for k in range(RPT // CH):
        pltpu.sync_copy(acc.at[pl.ds(src_base + CH * k, CH)], buf)
        pltpu.sync_copy(buf, out.at[pl.ds(dst_base + CH * k, CH)])


def _zero_rows(acc, buf, zrows, base):
    pltpu.sync_copy(zrows, buf)
    for k in range(RPT // CH):
        pltpu.sync_copy(buf, acc.at[pl.ds(base + CH * k, CH)])


def _scatter_loop(table, idx_v, dst_v, acc, bufs, gsems, ssems, nchunks):
    def gather(c, i):
        pltpu.async_copy(table.at[idx_v.at[c]], bufs[i], gsems[i])

    def wait_gather(c, i):
        pltpu.make_async_copy(table.at[idx_v.at[c]], bufs[i], gsems[i]).wait()

    def scatter(c, i):
        pltpu.async_copy(bufs[i], acc.at[dst_v.at[c]], ssems[i], add=True)

    def wait_scatter(c, i):
        pltpu.make_async_copy(bufs[i], acc.at[dst_v.at[c]], ssems[i]).wait()

    gather(0, 0)
    gather(1, 1)
    for c in range(4):
        wait_gather(c, c)
        scatter(c, c)
        if c < 2:
            gather(c + 2, c + 2)
        else:
            wait_scatter(c - 2, c - 2)
            gather(c + 2, c - 2)

    def body(j, carry):
        for i in range(4):
            c = 4 * j + i
            wait_gather(c, i)
            scatter(c, i)
            i2 = (i + 2) % 4
            wait_scatter(c - 2, i2)

            @pl.when(c + 2 < nchunks)
            def _():
                gather(c + 2, i2)

        return carry

    lax.fori_loop(1, nchunks // 4, body, 0)
    wait_scatter(nchunks - 2, 2)
    wait_scatter(nchunks - 1, 3)


@functools.cache
def _sc_segment_sum_kernel():
    @functools.partial(
        pl.kernel,
        out_type=jax.ShapeDtypeStruct((2 * ROWS, DH), jnp.float32),
        mesh=_mesh(),
        compiler_params=pltpu.CompilerParams(use_tc_tiling_on_sc=False),
        scratch_types=[
            pltpu.VMEM((CPT, CH), jnp.int32),
            pltpu.VMEM((CPT, CH), jnp.int32),
            [pltpu.VMEM((CH, DH), jnp.float32) for _ in range(4)],
            [pltpu.SemaphoreType.DMA for _ in range(4)],
            [pltpu.SemaphoreType.DMA for _ in range(4)],
            pltpu.VMEM_SHARED((ROWS, DH), jnp.float32),
        ],
    )
    def _sc_segment_sum(hflat, srcm, dstm, zrows, out,
                        src_v, dst_v, bufs, gsems, ssems, acc):
        cid = lax.axis_index("c")
        sid = lax.axis_index("s")
        pltpu.sync_copy(srcm.at[cid * NS + sid], src_v)
        pltpu.sync_copy(dstm.at[sid], dst_v)
        base = sid * RPT
        _zero_rows(acc, bufs[0], zrows, base)
        plsc.subcore_barrier()
        _scatter_loop(hflat, src_v, dst_v, acc, bufs, gsems, ssems, CPT)
        plsc.subcore_barrier()
        _writeout(acc, bufs[0], out, base, cid * ROWS + base)

    return _sc_segment_sum


@functools.cache
def _sc_combo_count_kernel():
    @functools.partial(
        pl.kernel,
        out_type=jax.ShapeDtypeStruct((2 * ROWS, 16), jnp.float32),
        mesh=_mesh(),
        compiler_params=pltpu.CompilerParams(use_tc_tiling_on_sc=False),
        scratch_types=[
            pltpu.VMEM((CPTC, CH), jnp.int32),
            pltpu.VMEM((CPTC, CH), jnp.int32),
            [pltpu.VMEM((CH, 16), jnp.float32) for _ in range(4)],
            [pltpu.SemaphoreType.DMA for _ in range(4)],
            [pltpu.SemaphoreType.DMA for _ in range(4)],
            pltpu.VMEM_SHARED((ROWS, 16), jnp.float32),
        ],
    )
    def _sc_combo_count(eye16, csl, dstc, zrows16, out,
                        c_v, dst_v, bufs, gsems, ssems, acc):
        cid = lax.axis_index("c")
        sid = lax.axis_index("s")
        w = cid * NS + sid
        pltpu.sync_copy(csl.at[w], c_v)
        pltpu.sync_copy(dstc.at[w], dst_v)
        base = sid * RPT
        _zero_rows(acc, bufs[0], zrows16, base)
        plsc.subcore_barrier()
        _scatter_loop(eye16, c_v, dst_v, acc, bufs, gsems, ssems, CPTC)
        plsc.subcore_barrier()
        _writeout(acc, bufs[0], out, base, cid * ROWS + base)

    return _sc_combo_count


BN = 2000


def _embed_body(p_ref, t_ref, out_ref):
    p = p_ref[...]
    for q in range(NQ):
        out_ref[q] = jnp.dot(p, t_ref[q], preferred_element_type=jnp.float32)


def _tc_embed(p9, t16q):
    return pl.pallas_call(
        _embed_body,
        grid=(N // BN,),
        in_specs=[
            pl.BlockSpec((BN, 16), lambda i: (i, 0)),
            pl.BlockSpec((NQ, 16, DH), lambda i: (0, 0, 0)),
        ],
        out_specs=pl.BlockSpec((NQ, BN, DH), lambda i: (0, i, 0)),
        out_shape=jax.ShapeDtypeStruct((NQ, N, DH), jnp.float32),
    )(p9, t16q)


def _layer_body(last, alo_ref, ahi_ref, h_ref, cnt_ref, eq_ref, esl_ref,
                w1_ref, b1_ref, w2_ref, b2_ref, out_ref):
    ct = cnt_ref[0] + cnt_ref[1]
    z = b1_ref[0:1, :]
    for q in range(NQ):
        a = alo_ref[q] if q < 2 else ahi_ref[q - 2]
        aq = (a + h_ref[q] + jnp.dot(ct, eq_ref[q]) + esl_ref[q, 0:1, :])
        z = z + jnp.dot(aq, w1_ref[q], preferred_element_type=jnp.float32)
    z = jnp.maximum(z, 0.0)
    for q in range(NQ):
        o = jnp.dot(z, w2_ref[q], preferred_element_type=jnp.float32) + b2_ref[q, 0:1, :]
        if not last:
            o = jnp.maximum(o, 0.0)
        out_ref[q] = o


def _tc_layer(last, alo, ahi, h4, cnt2, eq, esl, w1s, b1t, w2s, b2s):
    return pl.pallas_call(
        functools.partial(_layer_body, last),
        grid=(N // BN,),
        in_specs=[
            pl.BlockSpec((2, BN, DH), lambda i: (0, i, 0)),
            pl.BlockSpec((2, BN, DH), lambda i: (0, i, 0)),
            pl.BlockSpec((NQ, BN, DH), lambda i: (0, i, 0)),
            pl.BlockSpec((2, BN, 16), lambda i: (0, i, 0)),
            pl.BlockSpec((NQ, 16, DH), lambda i: (0, 0, 0)),
            pl.BlockSpec((NQ, 8, DH), lambda i: (0, 0, 0)),
            pl.BlockSpec((NQ, DH, 2 * D), lambda i: (0, 0, 0)),
            pl.BlockSpec((8, 2 * D), lambda i: (0, 0)),
            pl.BlockSpec((NQ, 2 * D, DH), lambda i: (0, 0, 0)),
            pl.BlockSpec((NQ, 8, DH), lambda i: (0, 0, 0)),
        ],
        out_specs=pl.BlockSpec((NQ, BN, DH), lambda i: (0, i, 0)),
        out_shape=jax.ShapeDtypeStruct((NQ, N, DH), jnp.float32),
    )(alo, ahi, h4, cnt2, eq, esl, w1s, b1t, w2s, b2s)


def _pool_body(h_ref, p_ref, wf_ref, bf_ref, out_ref):
    dn = (((0,), (0,)), ((), ()))
    acc = bf_ref[0:1, :]
    for q in range(NQ):
        hp = lax.dot_general(p_ref[...], h_ref[q], dn,
                             preferred_element_type=jnp.float32)
        acc = acc + jnp.dot(hp, wf_ref[q], preferred_element_type=jnp.float32)
    out_ref[...] = acc


def _tc_pool(h4, p64, wfs, bf8):
    return pl.pallas_call(
        _pool_body,
        out_shape=jax.ShapeDtypeStruct((G, FD), jnp.float32),
    )(h4, p64, wfs, bf8)


def _quarters(m, axis):
    padded = NQ * DH
    pad_widths = [(0, 0)] * m.ndim
    pad_widths[axis] = (0, padded - m.shape[axis])
    mp = jnp.pad(m, pad_widths)
    parts = [lax.slice_in_dim(mp, q * DH, (q + 1) * DH, axis=axis)
             for q in range(NQ)]
    return jnp.stack(parts)


def kernel(x, edge_index, edge_attr, batch, x_emb1, x_emb2, ee1, ee2,
           W1, b1, W2, b2, gamma, beta, Wf, bf):
    f32 = jnp.float32

    src = edge_index[0].astype(jnp.int32)
    dst = edge_index[1].astype(jnp.int32)
    pad = EP - E
    srcp = jnp.concatenate([src, jnp.zeros((pad,), jnp.int32)])
    dstp = jnp.concatenate([dst, jnp.full((pad,), DUMMY, jnp.int32)])
    srcm = [jnp.stack([srcp + 2 * p * N, srcp + (2 * p + 1) * N])
            .reshape(2 * NS, CPT, CH) for p in range(2)]
    dstm = dstp.reshape(NS, CPT, CH)
    c = (edge_attr[:, 0] * 3 + edge_attr[:, 1]).astype(jnp.int32)
    csl = jnp.concatenate([c, jnp.zeros((pad,), jnp.int32)]).reshape(NC * NS, CPTC, CH)
    dstc = dstp.reshape(NC * NS, CPTC, CH)

    zrows = jnp.zeros((CH, DH), f32)
    zrows16 = jnp.zeros((CH, 16), f32)
    eye16 = jnp.eye(16, dtype=f32)

    t9 = (x_emb1[:3][:, None, :] + x_emb2[None, :, :]).reshape(9, D)
    t16q = _quarters(jnp.concatenate([t9, jnp.zeros((7, D), f32)]), axis=1)
    idx9 = x[:, 0] * 3 + x[:, 1]
    p9 = (idx9[:, None] == jnp.arange(16)[None, :]).astype(f32)

    s = gamma / jnp.sqrt(1.0 + 1e-5)
    w2_eff = W2 * s[:, None, :]
    b2_eff = b2 * s + beta

    def tile8(v):
        return jnp.tile(v[None, :], (8, 1))

    p64 = (batch[:, None] == jnp.arange(G)[None, :]).astype(f32)
    wfs = _quarters(Wf, axis=0)
    bf8 = tile8(bf)

    h4 = _tc_embed(p9, t16q)
    cnt_raw = _sc_combo_count_kernel()(eye16, csl, dstc, zrows16)
    cnt2 = cnt_raw.reshape(2, ROWS, 16)

    for l in range(L):
        etab = (ee1[l][:3][:, None, :] + ee2[l][None, :, :]).reshape(9, D)
        eq = _quarters(jnp.concatenate([etab, jnp.zeros((7, D), f32)]), axis=1)
        esl = _quarters(tile8(ee1[l][4] + ee2[l][0]), axis=1)
        w1s = _quarters(W1[l], axis=0)
        b1t = tile8(b1[l])
        w2s = _quarters(w2_eff[l], axis=1)
        b2s = _quarters(tile8(b2_eff[l]), axis=1)

        hflat = h4.reshape(NQ * N, DH)
        alo = _sc_segment_sum_kernel()(hflat, srcm[0], dstm, zrows)
        ahi = _sc_segment_sum_kernel()(hflat, srcm[1], dstm, zrows)
        h4 = _tc_layer(l == L - 1, alo.reshape(2, ROWS, DH),
                       ahi.reshape(2, ROWS, DH), h4, cnt2,
                       eq, esl, w1s, b1t, w2s, b2s)

    return _tc_pool(h4, p64, wfs, bf8)


# speedup vs baseline: 1.1941x; 1.1941x over previous
import functools

import jax
import jax.numpy as jnp
from jax import lax
from jax.experimental import pallas as pl
from jax.experimental.pallas import tpu as pltpu
from jax.experimental.pallas import tpu_sc as plsc

N = 10000
E = 160000
D = 300
L = 5
G = 64
FD = 512

NC = 2
NS = 16
DH = 80
NQ = 4
CH = 128
CPT = 80
CPTC = 40
EP = NS * CPT * CH
ROWS = 10240
RPT = ROWS // NS
DUMMY = N + 16


@functools.cache
def _mesh():
    return plsc.VectorSubcoreMesh(core_axis_name="c", subcore_axis_name="s",
                                  num_cores=NC, num_subcores=NS)


def _writeout(acc, buf, out, src_base, dst_base):
    for k in range(RPT // CH):
        pltpu.sync_copy(acc.at[pl.ds(src_base + CH * k, CH)], buf)
        pltpu.sync_copy(buf, out.at[pl.ds(dst_base + CH * k, CH)])


def _zero_rows(acc, buf, zrows, base):
    pltpu.sync_copy(zrows, buf)
    for k in range(RPT // CH):
        pltpu.sync_copy(buf, acc.at[pl.ds(base + CH * k, CH)])


def _scatter_loop(table, idx_v, dst_v, acc, bufs, gsems, ssems, nchunks):
    def gather(c, i):
        pltpu.async_copy(table.at[idx_v.at[c]], bufs[i], gsems[i])

    def wait_gather(c, i):
        pltpu.make_async_copy(table.at[idx_v.at[c]], bufs[i], gsems[i]).wait()

    def scatter(c, i):
        pltpu.async_copy(bufs[i], acc.at[dst_v.at[c]], ssems[i], add=True)

    def wait_scatter(c, i):
        pltpu.make_async_copy(bufs[i], acc.at[dst_v.at[c]], ssems[i]).wait()

    gather(0, 0)
    gather(1, 1)
    for c in range(4):
        wait_gather(c, c)
        scatter(c, c)
        if c < 2:
            gather(c + 2, c + 2)
        else:
            wait_scatter(c - 2, c - 2)
            gather(c + 2, c - 2)

    def body(j, carry):
        for i in range(4):
            c = 4 * j + i
            wait_gather(c, i)
            scatter(c, i)
            i2 = (i + 2) % 4
            wait_scatter(c - 2, i2)

            @pl.when(c + 2 < nchunks)
            def _():
                gather(c + 2, i2)

        return carry

    lax.fori_loop(1, nchunks // 4, body, 0)
    wait_scatter(nchunks - 2, 2)
    wait_scatter(nchunks - 1, 3)


@functools.cache
def _sc_segment_sum_kernel():
    @functools.partial(
        pl.kernel,
        out_type=jax.ShapeDtypeStruct((2 * ROWS, DH), jnp.float32),
        mesh=_mesh(),
        compiler_params=pltpu.CompilerParams(use_tc_tiling_on_sc=False),
        scratch_types=[
            pltpu.VMEM((CPT, CH), jnp.int32),
            pltpu.VMEM((CPT, CH), jnp.int32),
            [pltpu.VMEM((CH, DH), jnp.float32) for _ in range(4)],
            [pltpu.SemaphoreType.DMA for _ in range(4)],
            [pltpu.SemaphoreType.DMA for _ in range(4)],
            pltpu.VMEM_SHARED((ROWS, DH), jnp.float32),
        ],
    )
    def _sc_segment_sum(hflat, srcm, dstm, zrows, out,
                        src_v, dst_v, bufs, gsems, ssems, acc):
        cid = lax.axis_index("c")
        sid = lax.axis_index("s")
        pltpu.sync_copy(srcm.at[cid * NS + sid], src_v)
        pltpu.sync_copy(dstm.at[sid], dst_v)
        base = sid * RPT
        _zero_rows(acc, bufs[0], zrows, base)
        plsc.subcore_barrier()
        _scatter_loop(hflat, src_v, dst_v, acc, bufs, gsems, ssems, CPT)
        plsc.subcore_barrier()
        _writeout(acc, bufs[0], out, base, cid * ROWS + base)

    return _sc_segment_sum


@functools.cache
def _sc_combo_count_kernel():
    @functools.partial(
        pl.kernel,
        out_type=jax.ShapeDtypeStruct((2 * ROWS, 16), jnp.float32),
        mesh=_mesh(),
        compiler_params=pltpu.CompilerParams(use_tc_tiling_on_sc=False),
        scratch_types=[
            pltpu.VMEM((CPTC, CH), jnp.int32),
            pltpu.VMEM((CPTC, CH), jnp.int32),
            [pltpu.VMEM((CH, 16), jnp.float32) for _ in range(4)],
            [pltpu.SemaphoreType.DMA for _ in range(4)],
            [pltpu.SemaphoreType.DMA for _ in range(4)],
            pltpu.VMEM_SHARED((ROWS, 16), jnp.float32),
        ],
    )
    def _sc_combo_count(eye16, csl, dstc, zrows16, out,
                        c_v, dst_v, bufs, gsems, ssems, acc):
        cid = lax.axis_index("c")
        sid = lax.axis_index("s")
        w = cid * NS + sid
        pltpu.sync_copy(csl.at[w], c_v)
        pltpu.sync_copy(dstc.at[w], dst_v)
        base = sid * RPT
        _zero_rows(acc, bufs[0], zrows16, base)
        plsc.subcore_barrier()
        _scatter_loop(eye16, c_v, dst_v, acc, bufs, gsems, ssems, CPTC)
        plsc.subcore_barrier()
        _writeout(acc, bufs[0], out, base, cid * ROWS + base)

    return _sc_combo_count


BN = 2000


def _embed_body(p_ref, t_ref, out_ref):
    p = p_ref[...]
    for q in range(NQ):
        out_ref[q] = jnp.dot(p, t_ref[q], preferred_element_type=jnp.float32)


def _tc_embed(p9, t16q):
    return pl.pallas_call(
        _embed_body,
        grid=(N // BN,),
        in_specs=[
            pl.BlockSpec((BN, 16), lambda i: (i, 0)),
            pl.BlockSpec((NQ, 16, DH), lambda i: (0, 0, 0)),
        ],
        out_specs=pl.BlockSpec((NQ, BN, DH), lambda i: (0, i, 0)),
        out_shape=jax.ShapeDtypeStruct((NQ, N, DH), jnp.float32),
    )(p9, t16q)


def _layer_body(last, alo_ref, ahi_ref, h_ref, cnt_ref, eq_ref, esl_ref,
                w1_ref, b1_ref, w2_ref, b2_ref, out_ref):
    ct = cnt_ref[0] + cnt_ref[1]
    z = b1_ref[0:1, :]
    for q in range(NQ):
        a = alo_ref[q] if q < 2 else ahi_ref[q - 2]
        aq = (a + h_ref[q] + jnp.dot(ct, eq_ref[q]) + esl_ref[q, 0:1, :])
        z = z + jnp.dot(aq, w1_ref[q], preferred_element_type=jnp.float32)
    z = jnp.maximum(z, 0.0)
    for q in range(NQ):
        o = jnp.dot(z, w2_ref[q], preferred_element_type=jnp.float32) + b2_ref[q, 0:1, :]
        if not last:
            o = jnp.maximum(o, 0.0)
        out_ref[q] = o


def _tc_layer(last, alo, ahi, h4, cnt2, eq, esl, w1s, b1t, w2s, b2s):
    return pl.pallas_call(
        functools.partial(_layer_body, last),
        grid=(N // BN,),
        in_specs=[
            pl.BlockSpec((2, BN, DH), lambda i: (0, i, 0)),
            pl.BlockSpec((2, BN, DH), lambda i: (0, i, 0)),
            pl.BlockSpec((NQ, BN, DH), lambda i: (0, i, 0)),
            pl.BlockSpec((2, BN, 16), lambda i: (0, i, 0)),
            pl.BlockSpec((NQ, 16, DH), lambda i: (0, 0, 0)),
            pl.BlockSpec((NQ, 8, DH), lambda i: (0, 0, 0)),
            pl.BlockSpec((NQ, DH, 2 * D), lambda i: (0, 0, 0)),
            pl.BlockSpec((8, 2 * D), lambda i: (0, 0)),
            pl.BlockSpec((NQ, 2 * D, DH), lambda i: (0, 0, 0)),
            pl.BlockSpec((NQ, 8, DH), lambda i: (0, 0, 0)),
        ],
        out_specs=pl.BlockSpec((NQ, BN, DH), lambda i: (0, i, 0)),
        out_shape=jax.ShapeDtypeStruct((NQ, N, DH), jnp.float32),
    )(alo, ahi, h4, cnt2, eq, esl, w1s, b1t, w2s, b2s)


def _pool_body(h_ref, p_ref, wf_ref, bf_ref, out_ref):
    dn = (((0,), (0,)), ((), ()))
    acc = bf_ref[0:1, :]
    for q in range(NQ):
        hp = lax.dot_general(p_ref[...], h_ref[q], dn,
                             preferred_element_type=jnp.float32)
        acc = acc + jnp.dot(hp, wf_ref[q], preferred_element_type=jnp.float32)
    out_ref[...] = acc


def _tc_pool(h4, p64, wfs, bf8):
    return pl.pallas_call(
        _pool_body,
        out_shape=jax.ShapeDtypeStruct((G, FD), jnp.float32),
    )(h4, p64, wfs, bf8)


def _quarters(m, axis):
    padded = NQ * DH
    pad_widths = [(0, 0)] * m.ndim
    pad_widths[axis] = (0, padded - m.shape[axis])
    mp = jnp.pad(m, pad_widths)
    parts = [lax.slice_in_dim(mp, q * DH, (q + 1) * DH, axis=axis)
             for q in range(NQ)]
    return jnp.stack(parts)


def kernel(x, edge_index, edge_attr, batch, x_emb1, x_emb2, ee1, ee2,
           W1, b1, W2, b2, gamma, beta, Wf, bf):
    f32 = jnp.float32

    src = edge_index[0].astype(jnp.int32)
    dst = edge_index[1].astype(jnp.int32)
    pad = EP - E
    srcp = jnp.concatenate([src, jnp.zeros((pad,), jnp.int32)])
    dstp = jnp.concatenate([dst, jnp.full((pad,), DUMMY, jnp.int32)])
    srcm = [jnp.stack([srcp + 2 * p * N, srcp + (2 * p + 1) * N])
            .reshape(2 * NS, CPT, CH) for p in range(2)]
    dstm = dstp.reshape(NS, CPT, CH)
    c = (edge_attr[:, 0] * 3 + edge_attr[:, 1]).astype(jnp.int32)
    cp = jnp.concatenate([c, jnp.zeros((pad,), jnp.int32)])
    cp = cp + 16 * (jnp.arange(EP, dtype=jnp.int32) % 512)
    csl = cp.reshape(NC * NS, CPTC, CH)
    dstc = dstp.reshape(NC * NS, CPTC, CH)

    zrows = jnp.zeros((CH, DH), f32)
    zrows16 = jnp.zeros((CH, 16), f32)
    eye16 = jnp.tile(jnp.eye(16, dtype=f32), (512, 1))

    t9 = (x_emb1[:3][:, None, :] + x_emb2[None, :, :]).reshape(9, D)
    t16q = _quarters(jnp.concatenate([t9, jnp.zeros((7, D), f32)]), axis=1)
    idx9 = x[:, 0] * 3 + x[:, 1]
    p9 = (idx9[:, None] == jnp.arange(16)[None, :]).astype(f32)

    s = gamma / jnp.sqrt(1.0 + 1e-5)
    w2_eff = W2 * s[:, None, :]
    b2_eff = b2 * s + beta

    def tile8(v):
        return jnp.tile(v[None, :], (8, 1))

    p64 = (batch[:, None] == jnp.arange(G)[None, :]).astype(f32)
    wfs = _quarters(Wf, axis=0)
    bf8 = tile8(bf)

    h4 = _tc_embed(p9, t16q)
    cnt_raw = _sc_combo_count_kernel()(eye16, csl, dstc, zrows16)
    cnt2 = cnt_raw.reshape(2, ROWS, 16)

    for l in range(L):
        etab = (ee1[l][:3][:, None, :] + ee2[l][None, :, :]).reshape(9, D)
        eq = _quarters(jnp.concatenate([etab, jnp.zeros((7, D), f32)]), axis=1)
        esl = _quarters(tile8(ee1[l][4] + ee2[l][0]), axis=1)
        w1s = _quarters(W1[l], axis=0)
        b1t = tile8(b1[l])
        w2s = _quarters(w2_eff[l], axis=1)
        b2s = _quarters(tile8(b2_eff[l]), axis=1)

        hflat = h4.reshape(NQ * N, DH)
        alo = _sc_segment_sum_kernel()(hflat, srcm[0], dstm, zrows)
        ahi = _sc_segment_sum_kernel()(hflat, srcm[1], dstm, zrows)
        h4 = _tc_layer(l == L - 1, alo.reshape(2, ROWS, DH),
                       ahi.reshape(2, ROWS, DH), h4, cnt2,
                       eq, esl, w1s, b1t, w2s, b2s)

    return _tc_pool(h4, p64, wfs, bf8)
